# baseline (speedup 1.0000x reference)
import functools

import jax
import jax.numpy as jnp
import numpy as np
from jax.experimental import pallas as pl
from jax.experimental.pallas import tpu as pltpu

_NS = 64
_NI = 128
_H = 128
_NEAR = 0.1
_FAR = 1000.0
_OUT = 3 + 1 + 1 + (_NS + _NI) + 3
_BIG = np.int32(0x7FFFFFFF)


def _shift_r(x, k, fill):
    b = x.shape[0]
    f = jnp.full((b, k), fill, x.dtype)
    return jnp.concatenate([f, x[:, : x.shape[1] - k]], axis=1)


def _shift_l(x, k, fill):
    b = x.shape[0]
    f = jnp.full((b, k), fill, x.dtype)
    return jnp.concatenate([x[:, k:], f], axis=1)


def _body(o_ref, d_ref, dens_ref, col_ref, tv_ref, dists_ref, kurev_ref,
          r_ref, s_ref, lx_ref, li_ref, out_ref):
    f32 = jnp.float32
    dens = dens_ref[:, :]
    tv = tv_ref[:, :]
    dists = dists_ref[:, :]
    hp = jax.lax.Precision.HIGHEST

    alpha = 1.0 - jnp.exp(-dens * dists)
    lg = jnp.log(jnp.maximum(1.0 - alpha + 1e-10, 1e-10))
    tin = jnp.exp(jax.lax.dot_general(
        lg, lx_ref[:, :], (((1,), (0,)), ((), ())), precision=hp,
        preferred_element_type=f32))
    w = alpha * tin

    acc = jnp.sum(w, axis=1, keepdims=True)
    depth = jnp.sum(w * tv, axis=1, keepdims=True)

    w_rep = jax.lax.dot_general(
        w, r_ref[:, :], (((1,), (0,)), ((), ())), preferred_element_type=f32)
    rgb = jax.lax.dot_general(
        col_ref[:, :] * w_rep, s_ref[:, :], (((1,), (0,)), ((), ())),
        preferred_element_type=f32)
    rgb = rgb + (1.0 - acc)

    w2 = w + 1e-5
    pdf = w2 / jnp.sum(w2, axis=1, keepdims=True)
    cdf = jax.lax.dot_general(
        pdf, li_ref[:, :], (((1,), (0,)), ((), ())), precision=hp,
        preferred_element_type=f32)
    cdf_last = cdf[:, _NS - 1 : _NS]

    b = dens.shape[0]
    kc = jax.lax.shift_left(jax.lax.bitcast_convert_type(cdf, jnp.int32), 1)
    lo = jnp.concatenate(
        [jnp.zeros((b, 1), jnp.int32), kc,
         jnp.full((b, _H - 1 - _NS), _BIG, jnp.int32)], axis=1)
    hi = jnp.broadcast_to(kurev_ref[:, :], (b, _NI))

    lo, hi = jnp.minimum(lo, hi), jnp.maximum(lo, hi)
    lane = jax.lax.broadcasted_iota(jnp.int32, (1, _H), 1)
    for s in (64, 32, 16, 8, 4, 2, 1):
        msk = (lane & s) == 0
        lo = jnp.where(msk, jnp.minimum(lo, pltpu.roll(lo, _H - s, 1)),
                       jnp.maximum(lo, pltpu.roll(lo, s, 1)))
        hi = jnp.where(msk, jnp.minimum(hi, pltpu.roll(hi, _H - s, 1)),
                       jnp.maximum(hi, pltpu.roll(hi, s, 1)))

    cdf_lo = (lo & 1) == 0
    cdf_hi = (hi & 1) == 0
    v_lo = jax.lax.bitcast_convert_type(jax.lax.shift_right_logical(lo, 1), f32)
    v_hi = jax.lax.bitcast_convert_type(jax.lax.shift_right_logical(hi, 1), f32)

    lane_f = lane.astype(f32)
    cnt_lo = jnp.where(cdf_lo,
                       lane_f + 1.0 - jnp.ceil(v_lo * np.float32(_NI - 1)),
                       lane_f - jnp.floor(v_lo * np.float32(_NI - 1) + 0.5))
    cnt_hi = jnp.where(cdf_hi,
                       lane_f + 129.0 - jnp.ceil(v_hi * np.float32(_NI - 1)),
                       lane_f + 128.0 - jnp.floor(v_hi * np.float32(_NI - 1) + 0.5))

    cdfp = jnp.concatenate(
        [jnp.zeros((b, 1), f32), cdf,
         jnp.broadcast_to(cdf_last, (b, _H - 1 - _NS))], axis=1)

    def finish(v, is_cdf, cnt):
        cnt_i = cnt.astype(jnp.int32)
        i0 = jnp.clip(cnt_i - 1, 0, _H - 1)
        i1 = jnp.clip(cnt_i, 0, _H - 1)
        cdf_g0 = jnp.take_along_axis(cdfp, i0, axis=1)
        cdf_g1 = jnp.take_along_axis(cdfp, i1, axis=1)
        idx0 = jnp.minimum(cnt - 1.0, np.float32(_NS - 1))
        idx1 = jnp.minimum(cnt, np.float32(_NS - 1))
        bins_g0 = _NEAR + (_FAR - _NEAR) * (idx0 * np.float32(1.0 / (_NS - 1)))
        bins_g1 = _NEAR + (_FAR - _NEAR) * (idx1 * np.float32(1.0 / (_NS - 1)))
        denom = cdf_g1 - cdf_g0
        denom = jnp.where(denom < 1e-5, 1.0, denom)
        tt = (v - cdf_g0) / denom
        t_samp = bins_g0 + tt * (bins_g1 - bins_g0)
        return jnp.where(is_cdf, bins_g0, t_samp)

    vals_lo = finish(v_lo, cdf_lo, cnt_lo)
    vals_hi = finish(v_hi, cdf_hi, cnt_hi)
    t_comb = jnp.concatenate([vals_lo, vals_hi[:, : _NS + _NI - _H]], axis=1)

    mean_t = (jnp.sum(vals_lo, axis=1, keepdims=True) +
              jnp.sum(vals_hi[:, : _NS + _NI - _H], axis=1, keepdims=True)
              ) * np.float32(1.0 / (_NS + _NI))
    pos_mean = o_ref[:, :] + d_ref[:, :] * mean_t

    out_ref[:, :] = jnp.concatenate(
        [rgb, depth, acc, t_comb, pos_mean], axis=1)


@jax.jit
def _run(ray_origins, ray_directions, densities, colors192):
    n = densities.shape[0]
    blk = 4096
    f32 = jnp.float32

    t_lin = jnp.linspace(0.0, 1.0, _NS).astype(f32)
    tv = (_NEAR + (_FAR - _NEAR) * t_lin).reshape(1, _NS)
    dists = jnp.concatenate(
        [tv[0, 1:] - tv[0, :-1], jnp.full((1,), 1e10, f32)]).reshape(1, _NS)
    u = jnp.linspace(0.0, 1.0, _NI).astype(f32)
    ku = jax.lax.shift_left(jax.lax.bitcast_convert_type(u, jnp.int32), 1) | 1
    ku_rev = ku[::-1].reshape(1, _NI)

    r_np = np.zeros((_NS, 3 * _NS), np.float32)
    s_np = np.zeros((3 * _NS, 3), np.float32)
    for k in range(_NS):
        for c in range(3):
            r_np[k, 3 * k + c] = 1.0
            s_np[3 * k + c, c] = 1.0
    r_mat = jnp.asarray(r_np)
    s_mat = jnp.asarray(s_np)
    lx_mat = jnp.asarray(np.triu(np.ones((_NS, _NS), np.float32), 1))
    li_mat = jnp.asarray(np.triu(np.ones((_NS, _NS), np.float32), 0))

    grid = (n // blk,)
    row = lambda i: (i, 0)
    const = lambda i: (0, 0)
    return pl.pallas_call(
        _body,
        grid=grid,
        in_specs=[
            pl.BlockSpec((blk, 3), row),
            pl.BlockSpec((blk, 3), row),
            pl.BlockSpec((blk, _NS), row),
            pl.BlockSpec((blk, 3 * _NS), row),
            pl.BlockSpec((1, _NS), const),
            pl.BlockSpec((1, _NS), const),
            pl.BlockSpec((1, _NI), const),
            pl.BlockSpec((_NS, 3 * _NS), const),
            pl.BlockSpec((3 * _NS, 3), const),
            pl.BlockSpec((_NS, _NS), const),
            pl.BlockSpec((_NS, _NS), const),
        ],
        out_specs=pl.BlockSpec((blk, _OUT), row),
        out_shape=jax.ShapeDtypeStruct((n, _OUT), f32),
    )(ray_origins, ray_directions, densities, colors192,
      tv, dists, ku_rev, r_mat, s_mat, lx_mat, li_mat)


def kernel(ray_origins, ray_directions, densities, colors):
    n = densities.shape[0]
    return _run(ray_origins, ray_directions, densities,
                colors.reshape(n, 3 * _NS))


# speedup vs baseline: 1.0292x; 1.0292x over previous
import jax
import jax.numpy as jnp
import numpy as np
from jax.experimental import pallas as pl
from jax.experimental.pallas import tpu as pltpu

_NS = 64
_NI = 128
_H = 128
_NEAR = 0.1
_FAR = 1000.0
_OUT = 3 + 1 + 1 + (_NS + _NI) + 3
_BIG = np.int32(0x7FFFFFFF)


def _body(o_ref, d_ref, dens_ref, col_ref, tv_ref, dists_ref, kurev_ref,
          r_ref, s_ref, lx_ref, li_ref, out_ref):
    f32 = jnp.float32
    dens = dens_ref[:, :]
    tv = tv_ref[:, :]
    dists = dists_ref[:, :]
    hp = jax.lax.Precision.HIGHEST

    alpha = 1.0 - jnp.exp(-dens * dists)
    lg = jnp.log(jnp.maximum(1.0 - alpha + 1e-10, 1e-10))
    tin = jnp.exp(jax.lax.dot_general(
        lg, lx_ref[:, :], (((1,), (0,)), ((), ())), precision=hp,
        preferred_element_type=f32))
    w = alpha * tin

    acc = jnp.sum(w, axis=1, keepdims=True)
    depth = jnp.sum(w * tv, axis=1, keepdims=True)

    w_rep = jax.lax.dot_general(
        w, r_ref[:, :], (((1,), (0,)), ((), ())), preferred_element_type=f32)
    rgb = jax.lax.dot_general(
        col_ref[:, :] * w_rep, s_ref[:, :], (((1,), (0,)), ((), ())),
        preferred_element_type=f32)
    rgb = rgb + (1.0 - acc)

    w2 = w + 1e-5
    pdf = w2 / jnp.sum(w2, axis=1, keepdims=True)
    cdf = jax.lax.dot_general(
        pdf, li_ref[:, :], (((1,), (0,)), ((), ())), precision=hp,
        preferred_element_type=f32)
    cdf_last = cdf[:, _NS - 1 : _NS]

    b = dens.shape[0]
    kc = jax.lax.shift_left(jax.lax.bitcast_convert_type(cdf, jnp.int32), 1)
    lo = jnp.concatenate(
        [jnp.zeros((b, 1), jnp.int32), kc,
         jnp.full((b, _H - 1 - _NS), _BIG, jnp.int32)], axis=1)
    hi = jnp.broadcast_to(kurev_ref[:, :], (b, _NI))

    lo, hi = jnp.minimum(lo, hi), jnp.maximum(lo, hi)
    lane = jax.lax.broadcasted_iota(jnp.int32, (1, _H), 1)
    for s in (64, 32, 16, 8, 4, 2, 1):
        msk = (lane & s) == 0
        lo = jnp.where(msk, jnp.minimum(lo, pltpu.roll(lo, _H - s, 1)),
                       jnp.maximum(lo, pltpu.roll(lo, s, 1)))
        hi = jnp.where(msk, jnp.minimum(hi, pltpu.roll(hi, _H - s, 1)),
                       jnp.maximum(hi, pltpu.roll(hi, s, 1)))

    cdf_lo = (lo & 1) == 0
    cdf_hi = (hi & 1) == 0
    v_lo = jax.lax.bitcast_convert_type(jax.lax.shift_right_logical(lo, 1), f32)
    v_hi = jax.lax.bitcast_convert_type(jax.lax.shift_right_logical(hi, 1), f32)

    lane_f = lane.astype(f32)
    cnt_lo = jnp.where(cdf_lo,
                       lane_f + 1.0 - jnp.ceil(v_lo * np.float32(_NI - 1)),
                       lane_f - jnp.floor(v_lo * np.float32(_NI - 1) + 0.5))
    cnt_hi = jnp.where(cdf_hi,
                       lane_f + 129.0 - jnp.ceil(v_hi * np.float32(_NI - 1)),
                       lane_f + 128.0 - jnp.floor(v_hi * np.float32(_NI - 1) + 0.5))

    cdfp = jnp.concatenate(
        [jnp.zeros((b, 1), f32), cdf,
         jnp.broadcast_to(cdf_last, (b, _H - 1 - _NS))], axis=1)

    def finish(v, is_cdf, cnt):
        cnt_i = cnt.astype(jnp.int32)
        i0 = jnp.clip(cnt_i - 1, 0, _H - 1)
        i1 = jnp.clip(cnt_i, 0, _H - 1)
        cdf_g0 = jnp.take_along_axis(cdfp, i0, axis=1)
        cdf_g1 = jnp.take_along_axis(cdfp, i1, axis=1)
        idx0 = jnp.minimum(cnt - 1.0, np.float32(_NS - 1))
        idx1 = jnp.minimum(cnt, np.float32(_NS - 1))
        bins_g0 = _NEAR + (_FAR - _NEAR) * (idx0 * np.float32(1.0 / (_NS - 1)))
        bins_g1 = _NEAR + (_FAR - _NEAR) * (idx1 * np.float32(1.0 / (_NS - 1)))
        denom = cdf_g1 - cdf_g0
        denom = jnp.where(denom < 1e-5, 1.0, denom)
        tt = (v - cdf_g0) / denom
        t_samp = bins_g0 + tt * (bins_g1 - bins_g0)
        return jnp.where(is_cdf, bins_g0, t_samp)

    vals_lo = finish(v_lo, cdf_lo, cnt_lo)
    vals_hi = finish(v_hi, cdf_hi, cnt_hi)
    t_comb = jnp.concatenate([vals_lo, vals_hi[:, : _NS + _NI - _H]], axis=1)

    mean_t = (jnp.sum(vals_lo, axis=1, keepdims=True) +
              jnp.sum(vals_hi[:, : _NS + _NI - _H], axis=1, keepdims=True)
              ) * np.float32(1.0 / (_NS + _NI))
    pos_mean = o_ref[:, :] + d_ref[:, :] * mean_t

    out_ref[:, :] = jnp.concatenate(
        [rgb, depth, acc, t_comb, pos_mean], axis=1)


@jax.jit
def _run(ray_origins, ray_directions, densities, colors192):
    n = densities.shape[0]
    blk = 2048
    f32 = jnp.float32

    t_lin = jnp.linspace(0.0, 1.0, _NS).astype(f32)
    tv = (_NEAR + (_FAR - _NEAR) * t_lin).reshape(1, _NS)
    dists = jnp.concatenate(
        [tv[0, 1:] - tv[0, :-1], jnp.full((1,), 1e10, f32)]).reshape(1, _NS)
    u = jnp.linspace(0.0, 1.0, _NI).astype(f32)
    ku = jax.lax.shift_left(jax.lax.bitcast_convert_type(u, jnp.int32), 1) | 1
    ku_rev = ku[::-1].reshape(1, _NI)

    r_np = np.zeros((_NS, 3 * _NS), np.float32)
    s_np = np.zeros((3 * _NS, 3), np.float32)
    for k in range(_NS):
        for c in range(3):
            r_np[k, 3 * k + c] = 1.0
            s_np[3 * k + c, c] = 1.0
    r_mat = jnp.asarray(r_np)
    s_mat = jnp.asarray(s_np)
    lx_mat = jnp.asarray(np.triu(np.ones((_NS, _NS), np.float32), 1))
    li_mat = jnp.asarray(np.triu(np.ones((_NS, _NS), np.float32), 0))

    grid = (n // blk,)
    row = lambda i: (i, 0)
    const = lambda i: (0, 0)
    return pl.pallas_call(
        _body,
        grid=grid,
        in_specs=[
            pl.BlockSpec((blk, 3), row),
            pl.BlockSpec((blk, 3), row),
            pl.BlockSpec((blk, _NS), row),
            pl.BlockSpec((blk, 3 * _NS), row),
            pl.BlockSpec((1, _NS), const),
            pl.BlockSpec((1, _NS), const),
            pl.BlockSpec((1, _NI), const),
            pl.BlockSpec((_NS, 3 * _NS), const),
            pl.BlockSpec((3 * _NS, 3), const),
            pl.BlockSpec((_NS, _NS), const),
            pl.BlockSpec((_NS, _NS), const),
        ],
        out_specs=pl.BlockSpec((blk, _OUT), row),
        out_shape=jax.ShapeDtypeStruct((n, _OUT), f32),
    )(ray_origins, ray_directions, densities, colors192,
      tv, dists, ku_rev, r_mat, s_mat, lx_mat, li_mat)


def kernel(ray_origins, ray_directions, densities, colors):
    n = densities.shape[0]
    return _run(ray_origins, ray_directions, densities,
                colors.reshape(n, 3 * _NS))
